# baseline (speedup 1.0000x reference)
import functools
import math

import jax
import jax.numpy as jnp
import numpy as np
from jax import lax
from jax.experimental import pallas as pl
from jax.experimental.pallas import tpu as pltpu
from jax.experimental.pallas import tpu_sc as plsc

_VOCAB = 1000000
_D = 64
_BATCH = 16384
_SEQ = 50

_NW = 32
_ROWS = _BATCH * _SEQ
_PER_W = _ROWS // _NW
_XSTAGE = 100
_NSTAGE = _PER_W // (128 * _XSTAGE)
_CSEQ = 4
_C = _CSEQ * _SEQ
_NCHUNK = _PER_W // _C
_SUBS = (104, 96)
_SCALE = 8.0


def _pos_encoding():
    positions = np.arange(_SEQ)[:, np.newaxis]
    div_term = np.exp(np.arange(0, _D, 2) * -(np.log(10000.0) / _D))
    angle_rads = positions * div_term
    pe = np.zeros((_SEQ, _D), dtype=np.float32)
    pe[:, 0::2] = np.sin(angle_rads)
    pe[:, 1::2] = np.cos(angle_rads)
    return pe


_PE = _pos_encoding()


def _make_sc_kernel():
    mesh = plsc.VectorSubcoreMesh(core_axis_name="c", subcore_axis_name="s")

    @functools.partial(
        pl.kernel,
        out_type=jax.ShapeDtypeStruct((_ROWS // 2, 128), jnp.float32),
        mesh=mesh,
        compiler_params=pltpu.CompilerParams(use_tc_tiling_on_sc=False),
        scratch_types=[
            pltpu.VMEM((_XSTAGE, 128), jnp.int32),
            pltpu.VMEM((_PER_W,), jnp.int32),
            pltpu.VMEM((2, _C, 128), jnp.float32),
            pltpu.VMEM((_C // 2, 128), jnp.float32),
            pltpu.VMEM((_SEQ, _D), jnp.float32),
            pltpu.SemaphoreType.DMA,
            pltpu.SemaphoreType.DMA,
        ],
    )
    def sc_kernel(x_hbm, pe_hbm, table_hbm, out_hbm, xs_v, idx_v, rows_v, ob_v,
                  pe_v, g0, g1):
        wid = lax.axis_index("s") * 2 + lax.axis_index("c")

        pltpu.sync_copy(pe_hbm, pe_v)

        for p in range(_NSTAGE):
            pltpu.sync_copy(
                x_hbm.at[pl.ds((wid * _NSTAGE + p) * _XSTAGE, _XSTAGE)], xs_v
            )

            def repack_body(r, carry):
                for j in range(128 // 16):
                    idx_v[pl.ds(p * _XSTAGE * 128 + r * 128 + 16 * j, 16)] = (
                        xs_v[r, pl.ds(16 * j, 16)]
                    )
                return carry

            lax.fori_loop(0, _XSTAGE, repack_body, 0)

        sems = (g0, g1)

        def sub_copies(c, b):
            copies = []
            off = 0
            for sub in _SUBS:
                copies.append((
                    table_hbm.at[idx_v.at[pl.ds(c * _C + off, sub)]],
                    rows_v.at[b, pl.ds(off, sub)],
                ))
                off += sub
            return copies

        def fire(c, b):
            for src, dst in sub_copies(c, b):
                pltpu.async_copy(src, dst, sems[b])

        def finish(c, b):
            for src, dst in sub_copies(c, b):
                pltpu.make_async_copy(src, dst, sems[b]).wait()

            def seq_body(q, carry):
                def row_body(r, carry2):
                    i = q * _SEQ + r
                    half = i // 2
                    col0 = (i % 2) * _D
                    for j in range(_D // 16):
                        ob_v[half, pl.ds(col0 + 16 * j, 16)] = (
                            rows_v[b, i, pl.ds(16 * j, 16)] * _SCALE
                            + pe_v[r, pl.ds(16 * j, 16)]
                        )
                    return carry2

                lax.fori_loop(0, _SEQ, row_body, 0)
                return carry

            lax.fori_loop(0, _CSEQ, seq_body, 0)

            h0 = (wid * _NCHUNK + c) * (_C // 2)
            pltpu.sync_copy(ob_v, out_hbm.at[pl.ds(h0, _C // 2)])

        fire(0, 0)

        def loop_body(c2, carry):
            c0 = 2 * c2

            fire(c0 + 1, 1)
            finish(c0, 0)

            @pl.when(c0 + 2 < _NCHUNK)
            def _():
                fire(c0 + 2, 0)

            finish(c0 + 1, 1)
            return carry

        lax.fori_loop(0, _NCHUNK // 2, loop_body, 0)

    return sc_kernel


_sc_kernel = _make_sc_kernel()


@jax.jit
def kernel(x, table):
    x128 = x.reshape(_ROWS // 128, 128)
    table128 = jnp.pad(table, ((0, 0), (0, 128 - _D)))
    pe = jnp.asarray(_PE)
    out = _sc_kernel(x128, pe, table128)
    return out.reshape(_BATCH, _SEQ, _D)
